# baseline (speedup 1.0000x reference)
import functools

import jax
import jax.numpy as jnp
from jax import lax
from jax.experimental import pallas as pl
from jax.experimental.pallas import tpu as pltpu
from jax.experimental.pallas import tpu_sc as plsc

N_NODES = 100000
EMB_DIM = 128
PRIOR_DIM = 64
OUT_DIM = PRIOR_DIM + EMB_DIM
CHUNK = 128
PAIRS = CHUNK // 2
BUF_ROWS = CHUNK * OUT_DIM // 128
PRI_ROWS = CHUNK * PRIOR_DIM // 128
NUM_FULL = N_NODES // CHUNK
REM = N_NODES - NUM_FULL * CHUNK
N_PAD = (NUM_FULL + 1) * CHUNK
NW = 32
NMAX = (NUM_FULL + NW - 1) // NW
LAST_FULL_W = (NUM_FULL - 1) % NW


def _build_kernel():
    mesh = plsc.VectorSubcoreMesh(core_axis_name="c", subcore_axis_name="s")

    @functools.partial(
        pl.kernel,
        mesh=mesh,
        out_type=jax.ShapeDtypeStruct((N_NODES * OUT_DIM // 128, 128),
                                      jnp.float32),
        scratch_types=[
            pltpu.VMEM((CHUNK,), jnp.int32),
            pltpu.VMEM((CHUNK,), jnp.int32),
            pltpu.VMEM((BUF_ROWS, 128), jnp.float32),
            pltpu.VMEM((BUF_ROWS, 128), jnp.float32),
            pltpu.VMEM((PRI_ROWS, 128), jnp.float32),
            pltpu.VMEM((PRI_ROWS, 128), jnp.float32),
            pltpu.SemaphoreType.DMA,
            pltpu.SemaphoreType.DMA,
            pltpu.SemaphoreType.DMA,
            pltpu.SemaphoreType.DMA,
            pltpu.SemaphoreType.DMA,
            pltpu.SemaphoreType.DMA,
            pltpu.SemaphoreType.DMA,
            pltpu.SemaphoreType.DMA,
        ],
    )
    def k(feat_hbm, inv_hbm, tab_hbm, out_hbm,
          idx0, idx1, buf0, buf1, pri0, pri1,
          gs0, gs1, vs0, vs1, is0, is1, ws0, ws1):
        idx = (idx0, idx1)
        buf = (buf0, buf1)
        pri = (pri0, pri1)
        gsem = (gs0, gs1)
        vsem = (vs0, vs1)
        isem = (is0, is1)
        wsem = (ws0, ws1)
        cid = lax.axis_index("c")
        sid = lax.axis_index("s")
        wid = sid * 2 + cid
        n = jnp.where(wid <= LAST_FULL_W, NMAX, NMAX - 1)

        def issue_gather(s):
            pltpu.async_copy(tab_hbm.at[idx[s]],
                             buf[s].at[pl.ds(0, CHUNK), :], gsem[s])

        def wait_gather(s):
            pltpu.make_async_copy(tab_hbm.at[idx[s]],
                                  buf[s].at[pl.ds(0, CHUNK), :],
                                  gsem[s]).wait()

        def issue_pri(t, s):
            c = wid + NW * t
            pltpu.async_copy(inv_hbm.at[pl.ds(c * PRI_ROWS, PRI_ROWS), :],
                             pri[s], vsem[s])

        def wait_pri(s):
            pltpu.make_async_copy(inv_hbm.at[pl.ds(0, PRI_ROWS), :], pri[s],
                                  vsem[s]).wait()

        def issue_idx(t, s):
            pltpu.async_copy(feat_hbm.at[pl.ds((wid + NW * t) * CHUNK, CHUNK)],
                             idx[s], isem[s])

        def wait_idx(s):
            pltpu.make_async_copy(feat_hbm.at[pl.ds(0, CHUNK)], idx[s],
                                  isem[s]).wait()

        def issue_write(t, s):
            c = wid + NW * t
            pltpu.async_copy(buf[s],
                             out_hbm.at[pl.ds(c * BUF_ROWS, BUF_ROWS), :],
                             wsem[s])

        def wait_write(s):
            pltpu.make_async_copy(buf[s], out_hbm.at[pl.ds(0, BUF_ROWS), :],
                                  wsem[s]).wait()

        def merge(s, npairs):
            b, p = buf[s], pri[s]

            def pair(i, carry):
                kk = npairs - 1 - i
                e = 2 * kk
                for g in range(8):
                    b[3 * kk + 2, pl.ds(g * 16, 16)] = \
                        b[e + 1, pl.ds(g * 16, 16)]
                for g in range(4):
                    b[3 * kk + 1, pl.ds(g * 16, 16)] = \
                        b[e, pl.ds(PRIOR_DIM + g * 16, 16)]
                for g in range(4):
                    b[3 * kk + 1, pl.ds(PRIOR_DIM + g * 16, 16)] = \
                        p[kk, pl.ds(PRIOR_DIM + g * 16, 16)]
                for g in range(4):
                    b[3 * kk, pl.ds(PRIOR_DIM + g * 16, 16)] = \
                        b[e, pl.ds(g * 16, 16)]
                for g in range(4):
                    b[3 * kk, pl.ds(g * 16, 16)] = p[kk, pl.ds(g * 16, 16)]
                return carry

            lax.fori_loop(0, npairs, pair, 0, unroll=2)

        pltpu.sync_copy(feat_hbm.at[pl.ds(wid * CHUNK, CHUNK)], idx[0])
        issue_gather(0)
        issue_pri(0, 0)
        issue_idx(1, 1)

        def half(cur, t_cur):
            nxt = 1 - cur
            t_nxt = t_cur + 1

            @pl.when(t_nxt < n)
            def _():
                wait_idx(nxt)

                @pl.when(t_nxt >= 2)
                def _():
                    wait_write(nxt)

                issue_gather(nxt)
                issue_pri(t_nxt, nxt)

            @pl.when(t_cur < n)
            def _():
                wait_gather(cur)
                wait_pri(cur)

                @pl.when(t_cur + 2 < n)
                def _():
                    issue_idx(t_cur + 2, cur)

                merge(cur, PAIRS)
                issue_write(t_cur, cur)

        def body(p, carry):
            half(0, 2 * p)
            half(1, 2 * p + 1)
            return carry

        lax.fori_loop(0, (NMAX + 1) // 2, body, 0)

        wait_write(0)
        wait_write(1)

        @pl.when(wid == NW - 1)
        def _tail():
            tail_pairs = REM // 2
            tail_rows = REM * OUT_DIM // 128
            pltpu.sync_copy(feat_hbm.at[pl.ds(NUM_FULL * CHUNK, CHUNK)],
                            idx[0])
            issue_gather(0)
            pltpu.sync_copy(
                inv_hbm.at[pl.ds(NUM_FULL * PRI_ROWS, REM * PRIOR_DIM // 128),
                           :],
                pri[0].at[pl.ds(0, REM * PRIOR_DIM // 128), :])
            wait_gather(0)
            merge(0, tail_pairs)
            pltpu.sync_copy(buf[0].at[pl.ds(0, tail_rows), :],
                            out_hbm.at[pl.ds(NUM_FULL * BUF_ROWS, tail_rows),
                                       :])

    return k


_KERNEL = _build_kernel()


def kernel(feature, invariant_node_features, table):
    feat = feature.astype(jnp.int32)
    feat_pad = jnp.pad(feat, (0, N_PAD - N_NODES))
    inv2 = invariant_node_features.reshape(N_NODES * PRIOR_DIM // 128, 128)
    out2 = _KERNEL(feat_pad, inv2, table)
    return out2.reshape(N_NODES, OUT_DIM)


# speedup vs baseline: 2.5624x; 2.5624x over previous
import functools

import jax
import jax.numpy as jnp
from jax import lax
from jax.experimental import pallas as pl
from jax.experimental.pallas import tpu as pltpu
from jax.experimental.pallas import tpu_sc as plsc

N_NODES = 100000
EMB_DIM = 128
PRIOR_DIM = 64
OUT_DIM = PRIOR_DIM + EMB_DIM
CHUNK = 128
NUM_FULL = N_NODES // CHUNK
REM = N_NODES - NUM_FULL * CHUNK
N_PAD = (NUM_FULL + 1) * CHUNK
NW = 32
NMAX = (NUM_FULL + NW - 1) // NW
LAST_FULL_W = (NUM_FULL - 1) % NW

TC_BLOCK = 800


def _build_gather():
    mesh = plsc.VectorSubcoreMesh(core_axis_name="c", subcore_axis_name="s")

    @functools.partial(
        pl.kernel,
        mesh=mesh,
        out_type=jax.ShapeDtypeStruct((N_NODES, EMB_DIM), jnp.float32),
        scratch_types=[
            pltpu.VMEM((CHUNK,), jnp.int32),
            pltpu.VMEM((CHUNK,), jnp.int32),
            pltpu.VMEM((CHUNK, EMB_DIM), jnp.float32),
            pltpu.VMEM((CHUNK, EMB_DIM), jnp.float32),
            pltpu.SemaphoreType.DMA,
            pltpu.SemaphoreType.DMA,
            pltpu.SemaphoreType.DMA,
            pltpu.SemaphoreType.DMA,
            pltpu.SemaphoreType.DMA,
            pltpu.SemaphoreType.DMA,
        ],
    )
    def k(feat_hbm, tab_hbm, emb_hbm,
          idx0, idx1, buf0, buf1, gs0, gs1, is0, is1, ws0, ws1):
        idx = (idx0, idx1)
        buf = (buf0, buf1)
        gsem = (gs0, gs1)
        isem = (is0, is1)
        wsem = (ws0, ws1)
        cid = lax.axis_index("c")
        sid = lax.axis_index("s")
        wid = sid * 2 + cid
        n = jnp.where(wid <= LAST_FULL_W, NMAX, NMAX - 1)

        def issue_gather(s):
            pltpu.async_copy(tab_hbm.at[idx[s]], buf[s], gsem[s])

        def wait_gather(s):
            pltpu.make_async_copy(tab_hbm.at[idx[s]], buf[s], gsem[s]).wait()

        def issue_idx(t, s):
            pltpu.async_copy(feat_hbm.at[pl.ds((wid + NW * t) * CHUNK, CHUNK)],
                             idx[s], isem[s])

        def wait_idx(s):
            pltpu.make_async_copy(feat_hbm.at[pl.ds(0, CHUNK)], idx[s],
                                  isem[s]).wait()

        def issue_write(t, s):
            c = wid + NW * t
            pltpu.async_copy(buf[s], emb_hbm.at[pl.ds(c * CHUNK, CHUNK), :],
                             wsem[s])

        def wait_write(s):
            pltpu.make_async_copy(buf[s], emb_hbm.at[pl.ds(0, CHUNK), :],
                                  wsem[s]).wait()

        pltpu.sync_copy(feat_hbm.at[pl.ds(wid * CHUNK, CHUNK)], idx[0])
        issue_gather(0)
        issue_idx(1, 1)

        def half(cur, t_cur):
            nxt = 1 - cur
            t_nxt = t_cur + 1

            @pl.when(t_nxt < n)
            def _():
                wait_idx(nxt)

                @pl.when(t_nxt >= 2)
                def _():
                    wait_write(nxt)

                issue_gather(nxt)

            @pl.when(t_cur < n)
            def _():
                wait_gather(cur)

                @pl.when(t_cur + 2 < n)
                def _():
                    issue_idx(t_cur + 2, cur)

                issue_write(t_cur, cur)

        def body(p, carry):
            half(0, 2 * p)
            half(1, 2 * p + 1)
            return carry

        lax.fori_loop(0, (NMAX + 1) // 2, body, 0)

        wait_write(0)
        wait_write(1)

        @pl.when(wid == NW - 1)
        def _tail():
            base = NUM_FULL * CHUNK
            pltpu.sync_copy(feat_hbm.at[pl.ds(base, CHUNK)], idx[0])
            issue_gather(0)
            wait_gather(0)
            pltpu.sync_copy(buf[0].at[pl.ds(0, REM), :],
                            emb_hbm.at[pl.ds(base, REM), :])

    return k


_GATHER = _build_gather()


def _concat_body(inv_ref, emb_ref, out_ref):
    out_ref[:, :PRIOR_DIM] = inv_ref[...]
    out_ref[:, PRIOR_DIM:] = emb_ref[...]


_CONCAT = pl.pallas_call(
    _concat_body,
    grid=(N_NODES // TC_BLOCK,),
    in_specs=[
        pl.BlockSpec((TC_BLOCK, PRIOR_DIM), lambda i: (i, 0)),
        pl.BlockSpec((TC_BLOCK, EMB_DIM), lambda i: (i, 0)),
    ],
    out_specs=pl.BlockSpec((TC_BLOCK, OUT_DIM), lambda i: (i, 0)),
    out_shape=jax.ShapeDtypeStruct((N_NODES, OUT_DIM), jnp.float32),
    compiler_params=pltpu.CompilerParams(
        dimension_semantics=("arbitrary",)),
)


def kernel(feature, invariant_node_features, table):
    feat = feature.astype(jnp.int32)
    feat_pad = jnp.pad(feat, (0, N_PAD - N_NODES))
    emb = _GATHER(feat_pad, table)
    return _CONCAT(invariant_node_features, emb)


# speedup vs baseline: 6.6159x; 2.5819x over previous
import functools

import jax
import jax.numpy as jnp
from jax import lax
from jax.experimental import pallas as pl
from jax.experimental.pallas import tpu as pltpu
from jax.experimental.pallas import tpu_sc as plsc

N_NODES = 100000
EMB_DIM = 128
PRIOR_DIM = 64
OUT_DIM = PRIOR_DIM + EMB_DIM
CHUNK = 256
HALF = 128
NUM_FULL = N_NODES // CHUNK
REM = N_NODES - NUM_FULL * CHUNK
NW = 32
NMAX = (NUM_FULL + NW - 1) // NW
LAST_FULL_W = (NUM_FULL - 1) % NW

TC_BLOCK = 4096


def _build_gather():
    mesh = plsc.VectorSubcoreMesh(core_axis_name="c", subcore_axis_name="s")

    @functools.partial(
        pl.kernel,
        mesh=mesh,
        out_type=jax.ShapeDtypeStruct((N_NODES, EMB_DIM), jnp.float32),
        scratch_types=[
            pltpu.VMEM((CHUNK,), jnp.int32),
            pltpu.VMEM((CHUNK,), jnp.int32),
            pltpu.VMEM((CHUNK, EMB_DIM), jnp.float32),
            pltpu.VMEM((CHUNK, EMB_DIM), jnp.float32),
            pltpu.SemaphoreType.DMA,
            pltpu.SemaphoreType.DMA,
            pltpu.SemaphoreType.DMA,
            pltpu.SemaphoreType.DMA,
            pltpu.SemaphoreType.DMA,
            pltpu.SemaphoreType.DMA,
        ],
    )
    def k(feat_hbm, tab_hbm, emb_hbm,
          idx0, idx1, buf0, buf1, gs0, gs1, is0, is1, ws0, ws1):
        idx = (idx0, idx1)
        buf = (buf0, buf1)
        gsem = (gs0, gs1)
        isem = (is0, is1)
        wsem = (ws0, ws1)
        cid = lax.axis_index("c")
        sid = lax.axis_index("s")
        wid = sid * 2 + cid
        n = jnp.where(wid <= LAST_FULL_W, NMAX, NMAX - 1)

        def issue_gather(s):
            pltpu.async_copy(tab_hbm.at[idx[s].at[pl.ds(0, HALF)]],
                             buf[s].at[pl.ds(0, HALF), :], gsem[s])
            pltpu.async_copy(tab_hbm.at[idx[s].at[pl.ds(HALF, HALF)]],
                             buf[s].at[pl.ds(HALF, HALF), :], gsem[s])

        def wait_gather(s):
            pltpu.make_async_copy(tab_hbm.at[idx[s].at[pl.ds(0, HALF)]],
                                  buf[s].at[pl.ds(0, HALF), :],
                                  gsem[s]).wait()
            pltpu.make_async_copy(tab_hbm.at[idx[s].at[pl.ds(HALF, HALF)]],
                                  buf[s].at[pl.ds(HALF, HALF), :],
                                  gsem[s]).wait()

        def issue_idx(t, s):
            pltpu.async_copy(feat_hbm.at[pl.ds((wid + NW * t) * CHUNK, CHUNK)],
                             idx[s], isem[s])

        def wait_idx(s):
            pltpu.make_async_copy(feat_hbm.at[pl.ds(0, CHUNK)], idx[s],
                                  isem[s]).wait()

        def issue_write(t, s):
            c = wid + NW * t
            pltpu.async_copy(buf[s], emb_hbm.at[pl.ds(c * CHUNK, CHUNK), :],
                             wsem[s])

        def wait_write(s):
            pltpu.make_async_copy(buf[s], emb_hbm.at[pl.ds(0, CHUNK), :],
                                  wsem[s]).wait()

        pltpu.sync_copy(feat_hbm.at[pl.ds(wid * CHUNK, CHUNK)], idx[0])
        issue_gather(0)
        issue_idx(1, 1)

        def half(cur, t_cur):
            nxt = 1 - cur
            t_nxt = t_cur + 1

            @pl.when(t_nxt < n)
            def _():
                wait_idx(nxt)

                @pl.when(t_nxt >= 2)
                def _():
                    wait_write(nxt)

                issue_gather(nxt)

            @pl.when(t_cur < n)
            def _():
                wait_gather(cur)

                @pl.when(t_cur + 2 < n)
                def _():
                    issue_idx(t_cur + 2, cur)

                issue_write(t_cur, cur)

        def body(p, carry):
            half(0, 2 * p)
            half(1, 2 * p + 1)
            return carry

        lax.fori_loop(0, (NMAX + 1) // 2, body, 0)

        wait_write(0)
        wait_write(1)

        @pl.when(wid == NW - 1)
        def _tail():
            base = NUM_FULL * CHUNK
            pltpu.sync_copy(feat_hbm.at[pl.ds(base, HALF)],
                            idx[0].at[pl.ds(0, HALF)])
            pltpu.sync_copy(feat_hbm.at[pl.ds(base + HALF, REM - HALF)],
                            idx[0].at[pl.ds(HALF, REM - HALF)])
            pltpu.async_copy(tab_hbm.at[idx[0].at[pl.ds(0, HALF)]],
                             buf[0].at[pl.ds(0, HALF), :], gsem[0])
            pltpu.async_copy(tab_hbm.at[idx[0].at[pl.ds(HALF, REM - HALF)]],
                             buf[0].at[pl.ds(HALF, REM - HALF), :], gsem[0])
            pltpu.make_async_copy(tab_hbm.at[idx[0].at[pl.ds(0, HALF)]],
                                  buf[0].at[pl.ds(0, HALF), :],
                                  gsem[0]).wait()
            pltpu.make_async_copy(tab_hbm.at[idx[0].at[pl.ds(HALF,
                                                             REM - HALF)]],
                                  buf[0].at[pl.ds(HALF, REM - HALF), :],
                                  gsem[0]).wait()
            pltpu.sync_copy(buf[0].at[pl.ds(0, REM), :],
                            emb_hbm.at[pl.ds(base, REM), :])

    return k


_GATHER = _build_gather()


def _concat_body(inv_ref, emb_ref, out_ref):
    out_ref[:PRIOR_DIM, :] = inv_ref[...]
    out_ref[PRIOR_DIM:, :] = emb_ref[...].T


_CONCAT = pl.pallas_call(
    _concat_body,
    grid=((N_NODES + TC_BLOCK - 1) // TC_BLOCK,),
    in_specs=[
        pl.BlockSpec((PRIOR_DIM, TC_BLOCK), lambda i: (0, i)),
        pl.BlockSpec((TC_BLOCK, EMB_DIM), lambda i: (i, 0)),
    ],
    out_specs=pl.BlockSpec((OUT_DIM, TC_BLOCK), lambda i: (0, i)),
    out_shape=jax.ShapeDtypeStruct((OUT_DIM, N_NODES), jnp.float32),
    compiler_params=pltpu.CompilerParams(
        dimension_semantics=("arbitrary",)),
)


def kernel(feature, invariant_node_features, table):
    feat = feature.astype(jnp.int32)
    emb = _GATHER(feat, table)
    out_t = _CONCAT(invariant_node_features.T, emb)
    return out_t.T
